# Initial kernel scaffold; baseline (speedup 1.0000x reference)
import jax
import jax.numpy as jnp
from jax.experimental import pallas as pl


def kernel(x_user, x_spot, edge_index_user_spot, edge_index_spot_user):
    raise NotImplementedError("write your pallas kernel here")


# speedup vs baseline: 5.4640x; 5.4640x over previous
import functools

import jax
import jax.numpy as jnp
from jax import lax
from jax.experimental import pallas as pl
from jax.experimental.pallas import tpu as pltpu
from jax.experimental.pallas import tpu_sc as plsc

N = 10000
D = 256
E = 160000
NC = 2
NS = 16
L = 16
DH = D // NC
NPAD = 10240
R = NPAD // NS
K = 128
ET = 10240
EPAD = ET * NS
NCH = ET // K
RCH = R // K


def _rsqrt16(x):
    i = lax.bitcast_convert_type(x, jnp.int32)
    i = jnp.int32(0x5F3759DF) - lax.shift_right_logical(i, 1)
    y = lax.bitcast_convert_type(i, jnp.float32)
    for _ in range(3):
        y = y * (1.5 - 0.5 * x * y * y)
    return y


_mesh = plsc.VectorSubcoreMesh(core_axis_name="c", subcore_axis_name="s")


@functools.partial(
    pl.kernel,
    mesh=_mesh,
    compiler_params=pltpu.CompilerParams(needs_layout_passes=False),
    out_type=(
        jax.ShapeDtypeStruct((NPAD, D), jnp.float32),
        jax.ShapeDtypeStruct((NPAD, D), jnp.float32),
        jax.ShapeDtypeStruct((NC * NPAD, DH), jnp.float32),
    ),
    scratch_types=[
        pltpu.VMEM_SHARED((NPAD, DH), jnp.float32),
        pltpu.VMEM_SHARED((NPAD,), jnp.float32),
        pltpu.VMEM_SHARED((NPAD,), jnp.float32),
        pltpu.VMEM((K, DH), jnp.float32),
        pltpu.VMEM((K,), jnp.int32),
        pltpu.VMEM((K,), jnp.int32),
        pltpu.VMEM((K,), jnp.float32),
        pltpu.VMEM((R,), jnp.float32),
        pltpu.VMEM((R,), jnp.float32),
        pltpu.SemaphoreType.DMA,
    ],
)
def _gcn(xu, xs, es_us, ed_us, es_su, ed_su,
         out_u, out_s, scaled,
         acc, dsrc, ddst, rows, idx_s, idx_d, ones, rs_s, rs_d, sem):
    c = lax.axis_index("c")
    s = lax.axis_index("s")
    rbase = s * R
    ebase = s * ET
    coff = c * NPAD

    for jj in range(K // L):
        ones[pl.ds(jj * L, L)] = jnp.ones((L,), jnp.float32)

    for x_hbm, es_hbm, ed_hbm, out_hbm in (
        (xu, es_us, ed_us, out_s),
        (xs, es_su, ed_su, out_u),
    ):
        def zero_rows(i, _):
            for jj in range(DH // L):
                rows[i, pl.ds(jj * L, L)] = jnp.zeros((L,), jnp.float32)
            return 0
        lax.fori_loop(0, K, zero_rows, 0)

        def zero_rs(i, _):
            rs_s[pl.ds(pl.multiple_of(i * L, L), L)] = jnp.zeros((L,), jnp.float32)
            return 0
        lax.fori_loop(0, R // L, zero_rs, 0)
        pltpu.sync_copy(rs_s, dsrc.at[pl.ds(rbase, R)])
        pltpu.sync_copy(rs_s, ddst.at[pl.ds(rbase, R)])
        for t in range(RCH):
            pltpu.sync_copy(rows, acc.at[pl.ds(rbase + t * K, K)])
        plsc.subcore_barrier()

        def deg_body(j, _):
            off = pl.multiple_of(ebase + j * K, K)
            pltpu.sync_copy(es_hbm.at[pl.ds(off, K)], idx_s)
            pltpu.sync_copy(ed_hbm.at[pl.ds(off, K)], idx_d)
            pltpu.sync_copy(ones, dsrc.at[idx_s], add=True)
            pltpu.sync_copy(ones, ddst.at[idx_d], add=True)
            return 0
        lax.fori_loop(0, NCH, deg_body, 0)
        plsc.subcore_barrier()

        pltpu.sync_copy(dsrc.at[pl.ds(rbase, R)], rs_s)
        pltpu.sync_copy(ddst.at[pl.ds(rbase, R)], rs_d)

        def rs_body(i, _):
            sl = pl.ds(pl.multiple_of(i * L, L), L)
            rs_s[sl] = _rsqrt16(jnp.maximum(rs_s[sl], 1.0))
            rs_d[sl] = _rsqrt16(jnp.maximum(rs_d[sl], 1.0))
            return 0
        lax.fori_loop(0, R // L, rs_body, 0)

        for t in range(RCH):
            row0 = rbase + t * K
            pltpu.sync_copy(x_hbm.at[pl.ds(row0, K), pl.ds(c * DH, DH)], rows)

            def scale_body(i, _, _t=t):
                rsv = plsc.load_gather(
                    rs_s, [jnp.full((L,), _t * K + i, jnp.int32)])
                for jj in range(DH // L):
                    rows[i, pl.ds(jj * L, L)] = rows[i, pl.ds(jj * L, L)] * rsv
                return 0
            lax.fori_loop(0, K, scale_body, 0)
            pltpu.sync_copy(rows, scaled.at[pl.ds(coff + row0, K)])
        plsc.subcore_barrier()

        def main_body(j, _):
            off = pl.multiple_of(ebase + j * K, K)
            pltpu.sync_copy(es_hbm.at[pl.ds(off, K)], idx_s)
            pltpu.sync_copy(ed_hbm.at[pl.ds(off, K)], idx_d)
            for jj in range(K // L):
                sl = pl.ds(jj * L, L)
                idx_s[sl] = idx_s[sl] + coff
            pltpu.async_copy(scaled.at[idx_s], rows, sem).wait()
            pltpu.sync_copy(rows, acc.at[idx_d], add=True)
            return 0
        lax.fori_loop(0, NCH, main_body, 0)
        plsc.subcore_barrier()

        for t in range(RCH):
            row0 = rbase + t * K
            pltpu.sync_copy(acc.at[pl.ds(row0, K)], rows)

            def flush_body(i, _, _t=t):
                rsv = plsc.load_gather(
                    rs_d, [jnp.full((L,), _t * K + i, jnp.int32)])
                for jj in range(DH // L):
                    v = rows[i, pl.ds(jj * L, L)] * rsv
                    rows[i, pl.ds(jj * L, L)] = jnp.maximum(v, 0.0)
                return 0
            lax.fori_loop(0, K, flush_body, 0)
            pltpu.sync_copy(rows, out_hbm.at[pl.ds(row0, K), pl.ds(c * DH, DH)])
        plsc.subcore_barrier()


def kernel(x_user, x_spot, edge_index_user_spot, edge_index_spot_user):
    xu = jnp.pad(x_user.astype(jnp.float32), ((0, NPAD - N), (0, 0)))
    xs = jnp.pad(x_spot.astype(jnp.float32), ((0, NPAD - N), (0, 0)))
    epad = jnp.full((2, EPAD - E), NPAD - 1, jnp.int32)
    eus = jnp.concatenate(
        [edge_index_user_spot.astype(jnp.int32), epad], axis=1)
    esu = jnp.concatenate(
        [edge_index_spot_user.astype(jnp.int32), epad], axis=1)
    out_u, out_s, _ = _gcn(xu, xs, eus[0], eus[1], esu[0], esu[1])
    return (out_u[:N], out_s[:N])


# speedup vs baseline: 9.1932x; 1.6825x over previous
import functools

import jax
import jax.numpy as jnp
from jax import lax
from jax.experimental import pallas as pl
from jax.experimental.pallas import tpu as pltpu
from jax.experimental.pallas import tpu_sc as plsc

N = 10000
D = 256
E = 160000
NC = 2
NS = 16
L = 16
DH = D // NC
NPAD = 10240
R = NPAD // NS
K = 128
ET = 10240
EPAD = ET * NS
NCH = ET // K
NH = 2
HCH = NCH // NH
RCH = R // K
DEG_FIRE = 8


def _rsqrt16(x):
    i = lax.bitcast_convert_type(x, jnp.int32)
    i = jnp.int32(0x5F3759DF) - lax.shift_right_logical(i, 1)
    y = lax.bitcast_convert_type(i, jnp.float32)
    for _ in range(3):
        y = y * (1.5 - 0.5 * x * y * y)
    return y


_mesh = plsc.VectorSubcoreMesh(core_axis_name="c", subcore_axis_name="s")


@functools.partial(
    pl.kernel,
    mesh=_mesh,
    compiler_params=pltpu.CompilerParams(needs_layout_passes=False),
    out_type=(
        jax.ShapeDtypeStruct((NPAD, D), jnp.float32),
        jax.ShapeDtypeStruct((NPAD, D), jnp.float32),
        jax.ShapeDtypeStruct((NC * NPAD, DH), jnp.float32),
    ),
    scratch_types=[
        pltpu.VMEM_SHARED((NPAD, DH), jnp.float32),
        pltpu.VMEM_SHARED((NPAD,), jnp.float32),
        pltpu.VMEM_SHARED((NPAD,), jnp.float32),
        pltpu.VMEM((K, DH), jnp.float32),
        pltpu.VMEM((K, DH), jnp.float32),
        pltpu.VMEM((HCH, K), jnp.int32),
        pltpu.VMEM((HCH, K), jnp.int32),
        pltpu.VMEM((K,), jnp.float32),
        pltpu.VMEM((R,), jnp.float32),
        pltpu.VMEM((R,), jnp.float32),
        pltpu.SemaphoreType.DMA,
        pltpu.SemaphoreType.DMA,
        pltpu.SemaphoreType.DMA,
        pltpu.SemaphoreType.DMA,
    ],
)
def _gcn(xu, xs, es_us, ed_us, es_su, ed_su,
         out_u, out_s, scaled,
         acc, dsrc, ddst, rows0, rows1, isall, idall, ones, rs_s, rs_d,
         gsem0, gsem1, ssem0, ssem1):
    c = lax.axis_index("c")
    s = lax.axis_index("s")
    rbase = s * R
    coff = c * NPAD
    bufs = ((rows0, gsem0, ssem0), (rows1, gsem1, ssem1))

    for jj in range(K // L):
        ones[pl.ds(jj * L, L)] = jnp.ones((L,), jnp.float32)

    for x_hbm, es_hbm, ed_hbm, out_hbm in (
        (xu, es_us, ed_us, out_s),
        (xs, es_su, ed_su, out_u),
    ):
        def zero_rows(i, _):
            for jj in range(DH // L):
                rows0[i, pl.ds(jj * L, L)] = jnp.zeros((L,), jnp.float32)
            return 0
        lax.fori_loop(0, K, zero_rows, 0)

        def zero_rs(i, _):
            rs_s[pl.ds(pl.multiple_of(i * L, L), L)] = jnp.zeros((L,), jnp.float32)
            return 0
        lax.fori_loop(0, R // L, zero_rs, 0)

        zdescs = [
            pltpu.async_copy(rs_s, dsrc.at[pl.ds(rbase, R)], ssem0),
            pltpu.async_copy(rs_s, ddst.at[pl.ds(rbase, R)], ssem0),
        ]
        zdescs += [
            pltpu.async_copy(rows0, acc.at[pl.ds(rbase + t * K, K)], ssem0)
            for t in range(RCH)
        ]
        for dsc in zdescs:
            dsc.wait()
        plsc.subcore_barrier()

        for h in range(NH):
            d_is = pltpu.async_copy(
                es_hbm.at[s, pl.ds(h * HCH, HCH)], isall, gsem0)
            d_id = pltpu.async_copy(
                ed_hbm.at[s, pl.ds(h * HCH, HCH)], idall, gsem1)
            d_is.wait()
            d_id.wait()

            def deg_body(j3, _):
                descs = []
                for jj in range(DEG_FIRE):
                    j = j3 * DEG_FIRE + jj
                    descs.append(pltpu.async_copy(
                        ones, dsrc.at[isall.at[j]], gsem0, add=True))
                    descs.append(pltpu.async_copy(
                        ones, ddst.at[idall.at[j]], gsem1, add=True))
                for dsc in descs:
                    dsc.wait()
                return 0
            lax.fori_loop(0, HCH // DEG_FIRE, deg_body, 0)
        plsc.subcore_barrier()

        pltpu.sync_copy(dsrc.at[pl.ds(rbase, R)], rs_s)
        pltpu.sync_copy(ddst.at[pl.ds(rbase, R)], rs_d)

        def rs_body(i, _):
            sl = pl.ds(pl.multiple_of(i * L, L), L)
            rs_s[sl] = _rsqrt16(jnp.maximum(rs_s[sl], 1.0))
            rs_d[sl] = _rsqrt16(jnp.maximum(rs_d[sl], 1.0))
            return 0
        lax.fori_loop(0, R // L, rs_body, 0)

        loads = [None] * RCH
        stores = [None] * RCH

        def start_load(t):
            b, gsem, _ = bufs[t % 2]
            return pltpu.async_copy(
                x_hbm.at[pl.ds(rbase + t * K, K), pl.ds(c * DH, DH)], b, gsem)

        loads[0] = start_load(0)
        for t in range(RCH):
            b, _, ssem = bufs[t % 2]
            if t + 1 < RCH:
                if t - 1 >= 0:
                    stores[t - 1].wait()
                loads[t + 1] = start_load(t + 1)
            loads[t].wait()

            def scale_body(i, _, _t=t, _b=b):
                rsv = plsc.load_gather(
                    rs_s, [jnp.full((L,), _t * K + i, jnp.int32)])
                for jj in range(DH // L):
                    _b[i, pl.ds(jj * L, L)] = _b[i, pl.ds(jj * L, L)] * rsv
                return 0
            lax.fori_loop(0, K, scale_body, 0)
            stores[t] = pltpu.async_copy(
                b, scaled.at[pl.ds(coff + rbase + t * K, K)], ssem)
        for t in (RCH - 2, RCH - 1):
            stores[t].wait()
        plsc.subcore_barrier()

        for h in range(NH):
            d_is = pltpu.async_copy(
                es_hbm.at[s, pl.ds(h * HCH, HCH)], isall, gsem0)
            d_id = pltpu.async_copy(
                ed_hbm.at[s, pl.ds(h * HCH, HCH)], idall, gsem1)
            d_is.wait()
            d_id.wait()

            def shift_body(j, _):
                for jj in range(K // L):
                    sl = pl.ds(jj * L, L)
                    isall[j, sl] = isall[j, sl] + coff
                return 0
            lax.fori_loop(0, HCH, shift_body, 0)

            pltpu.make_async_copy(scaled.at[isall.at[0]], rows0, gsem0).start()
            pltpu.make_async_copy(scaled.at[isall.at[1]], rows1, gsem1).start()

            def main_body(j2, _):
                for b in (0, 1):
                    j = j2 * 2 + b
                    rows_b, gsem, ssem = bufs[b]
                    pltpu.make_async_copy(
                        scaled.at[isall.at[j]], rows_b, gsem).wait()
                    pltpu.async_copy(
                        rows_b, acc.at[idall.at[j]], ssem, add=True).wait()

                    @pl.when(j2 < HCH // 2 - 1)
                    def _():
                        pltpu.make_async_copy(
                            scaled.at[isall.at[j + 2]], rows_b, gsem).start()
                return 0
            lax.fori_loop(0, HCH // 2, main_body, 0)
        plsc.subcore_barrier()

        floads = [None] * RCH
        fstores = [None] * RCH

        def start_fload(t):
            b, gsem, _ = bufs[t % 2]
            return pltpu.async_copy(acc.at[pl.ds(rbase + t * K, K)], b, gsem)

        floads[0] = start_fload(0)
        for t in range(RCH):
            b, _, ssem = bufs[t % 2]
            if t + 1 < RCH:
                if t - 1 >= 0:
                    fstores[t - 1].wait()
                floads[t + 1] = start_fload(t + 1)
            floads[t].wait()

            def flush_body(i, _, _t=t, _b=b):
                rsv = plsc.load_gather(
                    rs_d, [jnp.full((L,), _t * K + i, jnp.int32)])
                for jj in range(DH // L):
                    v = _b[i, pl.ds(jj * L, L)] * rsv
                    _b[i, pl.ds(jj * L, L)] = jnp.maximum(v, 0.0)
                return 0
            lax.fori_loop(0, K, flush_body, 0)
            fstores[t] = pltpu.async_copy(
                b, out_hbm.at[pl.ds(rbase + t * K, K), pl.ds(c * DH, DH)], ssem)
        for t in (RCH - 2, RCH - 1):
            fstores[t].wait()
        plsc.subcore_barrier()


def kernel(x_user, x_spot, edge_index_user_spot, edge_index_spot_user):
    xu = jnp.pad(x_user.astype(jnp.float32), ((0, NPAD - N), (0, 0)))
    xs = jnp.pad(x_spot.astype(jnp.float32), ((0, NPAD - N), (0, 0)))
    epad = jnp.full((2, EPAD - E), NPAD - 1, jnp.int32)
    eus = jnp.concatenate(
        [edge_index_user_spot.astype(jnp.int32), epad], axis=1)
    esu = jnp.concatenate(
        [edge_index_spot_user.astype(jnp.int32), epad], axis=1)
    out_u, out_s, _ = _gcn(
        xu, xs,
        eus[0].reshape(NS, NCH, K), eus[1].reshape(NS, NCH, K),
        esu[0].reshape(NS, NCH, K), esu[1].reshape(NS, NCH, K))
    return (out_u[:N], out_s[:N])
